# baseline (speedup 1.0000x reference)
import numpy as np
import jax
import jax.numpy as jnp
from jax.experimental import pallas as pl
from jax.experimental.pallas import tpu as pltpu

VGG_CFG = [64, 64, 'M', 128, 128, 'M', 256, 256, 256, 'M',
           512, 512, 512, 'M', 512, 512, 512, 'M']


def _ceil_to(x, m):
    return -(-x // m) * m


def _border_mask(H, W, rows):
    Hp, Wp = H + 2, W + 2
    r = np.arange(Hp * Wp)
    i, j = r // Wp, r % Wp
    valid = (i >= 1) & (i <= H) & (j >= 1) & (j <= W)
    m = np.zeros((rows, 1), np.float32)
    m[:Hp * Wp, 0] = valid.astype(np.float32)
    return m


def _conv3x3(x_flat, w, b, H, W, relu=True):
    Hp, Wp = H + 2, W + 2
    R = Hp * Wp
    halo = Wp + 1
    Cin = x_flat.shape[1]
    Cout = w.shape[0]
    TM = min(_ceil_to(R, 8), 4096)
    n = -(-R // TM)
    SZ = _ceil_to(TM + 2 * halo, 8)

    x_ext = jnp.zeros(((n - 1) * TM + SZ, Cin), x_flat.dtype)
    x_ext = x_ext.at[halo:halo + R].set(x_flat)
    w9 = jnp.transpose(w, (2, 3, 1, 0)).reshape(9, Cin, Cout)
    mask = jnp.asarray(_border_mask(H, W, n * TM))
    offs = [dy * Wp + dx for dy in (0, 1, 2) for dx in (0, 1, 2)]

    def body(x_hbm, w_ref, b_ref, m_ref, o_ref, xs, sem):
        i = pl.program_id(0)
        cp = pltpu.make_async_copy(
            x_hbm.at[pl.ds(i * TM, SZ)], xs, sem)
        cp.start()
        cp.wait()
        acc = jnp.zeros((TM, Cout), jnp.float32)
        for t, off in enumerate(offs):
            acc += jnp.dot(xs[pl.ds(off, TM), :], w_ref[t],
                           preferred_element_type=jnp.float32)
        acc = acc + b_ref[0]
        if relu:
            acc = jnp.maximum(acc, 0.0)
        o_ref[...] = acc * m_ref[...]

    return pl.pallas_call(
        body,
        grid=(n,),
        in_specs=[
            pl.BlockSpec(memory_space=pl.ANY),
            pl.BlockSpec((9, Cin, Cout), lambda i: (0, 0, 0)),
            pl.BlockSpec((1, Cout), lambda i: (0, 0)),
            pl.BlockSpec((TM, 1), lambda i: (i, 0)),
        ],
        out_specs=pl.BlockSpec((TM, Cout), lambda i: (i, 0)),
        out_shape=jax.ShapeDtypeStruct((R, Cout), jnp.float32),
        scratch_shapes=[pltpu.VMEM((SZ, Cin), jnp.float32),
                        pltpu.SemaphoreType.DMA],
    )(x_ext, w9, b.reshape(1, Cout), mask)


def _pool_and_pad(y, H, W):
    C = y.shape[1]
    y3 = y.reshape(H + 2, W + 2, C)[1:H + 1, 1:W + 1]
    p = y3.reshape(H // 2, 2, W // 2, 2, C).max(axis=(1, 3))
    H2, W2 = H // 2, W // 2
    return jnp.pad(p, ((1, 1), (1, 1), (0, 0))).reshape((H2 + 2) * (W2 + 2), C)


def _heads(fm, ws, bs, wb, bb):
    P = fm.shape[0]

    def body(f_ref, ws_ref, bs_ref, wb_ref, bb_ref, os_ref, ob_ref):
        f = f_ref[...]
        os_ref[...] = jnp.dot(f, ws_ref[...],
                              preferred_element_type=jnp.float32) + bs_ref[0]
        ob_ref[...] = jnp.dot(f, wb_ref[...],
                              preferred_element_type=jnp.float32) + bb_ref[0]

    return pl.pallas_call(
        body,
        out_shape=(jax.ShapeDtypeStruct((P, 2), jnp.float32),
                   jax.ShapeDtypeStruct((P, 8), jnp.float32)),
    )(fm, ws, bs.reshape(1, -1), wb, bb.reshape(1, -1))


def kernel(batch_images, params):
    B, _, H, W = batch_images.shape
    x = jnp.transpose(batch_images[0], (1, 2, 0))
    x = jnp.pad(x, ((1, 1), (1, 1), (0, 5)))
    x = x.reshape((H + 2) * (W + 2), 8)

    li = 0
    for v in VGG_CFG:
        if v == 'M':
            x = _pool_and_pad(x, H, W)
            H //= 2
            W //= 2
        else:
            w = params['vgg_w%d' % li]
            b = params['vgg_b%d' % li]
            if li == 0:
                w = jnp.pad(w, ((0, 0), (0, 5), (0, 0), (0, 0)))
            x = _conv3x3(x, w, b, H, W)
            li += 1

    x = _conv3x3(x, params['common_w'], params['common_b'], H, W)
    C = x.shape[1]
    fm = x.reshape(H + 2, W + 2, C)[1:H + 1, 1:W + 1].reshape(H * W, C)

    ws = params['score_w'].reshape(2, C).T
    wb = params['bbox_w'].reshape(8, C).T
    score, bbox = _heads(fm, ws, params['score_b'], wb, params['bbox_b'])

    feature_map = fm.T.reshape(1, C, H, W)
    object_score = score.T.reshape(1, 2, H, W)
    bb = bbox.T.reshape(1, 8, H, W)
    bbox_regression = jnp.transpose(bb.reshape(1, 4, -1), (0, 2, 1))
    return feature_map, object_score, bbox_regression
